# baseline (speedup 1.0000x reference)
import jax
import jax.numpy as jnp
from jax.experimental import pallas as pl
from jax.experimental.pallas import tpu as pltpu

_CHUNK_ROWS = 512
_NBUF = 3


def _pipelined_scale_copy(w_hbm, o_hbm, vin, vout, in_sems, out_sems, *,
                          scale, n_chunks):
    def in_copy(i):
        slot = i % _NBUF
        return pltpu.make_async_copy(
            w_hbm.at[pl.ds(i * _CHUNK_ROWS, _CHUNK_ROWS), :],
            vin.at[slot],
            in_sems.at[slot],
        )

    def out_copy(i):
        slot = i % _NBUF
        return pltpu.make_async_copy(
            vout.at[slot],
            o_hbm.at[pl.ds(i * _CHUNK_ROWS, _CHUNK_ROWS), :],
            out_sems.at[slot],
        )

    for i in range(min(_NBUF, n_chunks)):
        in_copy(i).start()
    for i in range(n_chunks):
        slot = i % _NBUF
        in_copy(i).wait()
        if i >= _NBUF:
            out_copy(i - _NBUF).wait()
        vout[slot] = vin[slot] * scale
        out_copy(i).start()
        if i + _NBUF < n_chunks:
            in_copy(i + _NBUF).start()
    for i in range(max(0, n_chunks - _NBUF), n_chunks):
        out_copy(i).wait()


def kernel(x, emb_weight):
    seq_len = x.shape[1]
    max_seq, dim = emb_weight.shape
    assert seq_len <= max_seq
    assert seq_len % _CHUNK_ROWS == 0
    scale = dim ** (-0.5)
    n_chunks = seq_len // _CHUNK_ROWS
    import functools
    return pl.pallas_call(
        functools.partial(_pipelined_scale_copy, scale=scale,
                          n_chunks=n_chunks),
        in_specs=[pl.BlockSpec(memory_space=pl.ANY)],
        out_specs=pl.BlockSpec(memory_space=pl.ANY),
        out_shape=jax.ShapeDtypeStruct((seq_len, dim), emb_weight.dtype),
        scratch_shapes=[
            pltpu.VMEM((_NBUF, _CHUNK_ROWS, dim), emb_weight.dtype),
            pltpu.VMEM((_NBUF, _CHUNK_ROWS, dim), emb_weight.dtype),
            pltpu.SemaphoreType.DMA((_NBUF,)),
            pltpu.SemaphoreType.DMA((_NBUF,)),
        ],
        compiler_params=pltpu.CompilerParams(
            vmem_limit_bytes=100 * 1024 * 1024,
        ),
    )(emb_weight)
